# baseline (speedup 1.0000x reference)
import functools

import jax
import jax.numpy as jnp
from jax import lax
from jax.experimental import pallas as pl
from jax.experimental.pallas import tpu as pltpu
from jax.experimental.pallas import tpu_sc as plsc

BATCH = 4
SEQ = 8192
HIDDEN = 1024
NUM_WORKERS = 32
TOTAL = BATCH * SEQ
PER_WORKER = TOTAL // NUM_WORKERS
W = 56
TAIL_W = PER_WORKER - (PER_WORKER // W) * W
NBUF = 2
SKEW = NBUF - 1
NCH = PER_WORKER // W + 1
STEADY_END = NBUF + ((NCH - NBUF) // NBUF) * NBUF


def _gather_kernel(idx_hbm, table_hbm, out_hbm, idx_v, rows, gsems, wsems):
    wid = lax.axis_index("s") * 2 + lax.axis_index("c")
    base = wid * PER_WORKER

    pltpu.sync_copy(idx_hbm.at[wid], idx_v)

    def start_gather(j, b):
        pltpu.async_copy(table_hbm.at[idx_v.at[j]], rows.at[b], gsems.at[b])

    def wait_gather(b):
        pltpu.make_async_copy(table_hbm.at[idx_v.at[0]], rows.at[b],
                              gsems.at[b]).wait()

    def start_writeback(j, b, n=W):
        pltpu.async_copy(rows.at[b].at[pl.ds(0, n)],
                         out_hbm.at[pl.ds(base + j * W, n)], wsems.at[b])

    def wait_writeback(b, n=W):
        pltpu.make_async_copy(rows.at[b].at[pl.ds(0, n)],
                              out_hbm.at[pl.ds(base, n)], wsems.at[b]).wait()

    def slot(j, u, first=False, n=W):
        bg = (u + SKEW) % NBUF
        if not first:
            wait_writeback(bg)
        start_gather(j + SKEW, bg)
        wait_gather(u)
        start_writeback(j, u, n)

    for b in range(SKEW):
        start_gather(b, b)

    for u in range(NBUF):
        slot(u, u, first=(u == 0))

    @pl.loop(NBUF, STEADY_END, step=NBUF)
    def _(i):
        for u in range(NBUF):
            slot(i + u, u)

    for j in range(STEADY_END, NCH):
        slot(j, j % NBUF, n=(TAIL_W if j == NCH - 1 else W))

    wait_writeback((NCH - 1) % NBUF, TAIL_W)
    for t in range(SKEW):
        wait_gather((NCH + t) % NBUF)


def kernel(position_ids, table):
    ids = position_ids.reshape(NUM_WORKERS, PER_WORKER).astype(jnp.int32)
    ids = jnp.pad(ids, ((0, 0), (0, (NCH + SKEW) * W - PER_WORKER)))
    ids = ids.reshape(NUM_WORKERS, NCH + SKEW, W)

    mesh = plsc.VectorSubcoreMesh(core_axis_name="c", subcore_axis_name="s")

    run = functools.partial(
        pl.kernel,
        mesh=mesh,
        out_type=jax.ShapeDtypeStruct((TOTAL, HIDDEN), jnp.float32),
        scratch_types=[
            pltpu.VMEM((NCH + SKEW, W), jnp.int32),
            pltpu.VMEM((NBUF, W, HIDDEN), jnp.float32),
            pltpu.SemaphoreType.DMA((NBUF,)),
            pltpu.SemaphoreType.DMA((NBUF,)),
        ],
    )(_gather_kernel)

    out = run(ids, table)
    return out.reshape(BATCH, SEQ, HIDDEN)


# speedup vs baseline: 1.5352x; 1.5352x over previous
import functools

import jax
import jax.numpy as jnp
from jax import lax
from jax.experimental import pallas as pl
from jax.experimental.pallas import tpu as pltpu
from jax.experimental.pallas import tpu_sc as plsc

BATCH = 4
SEQ = 8192
HIDDEN = 1024
NUM_WORKERS = 32
TOTAL = BATCH * SEQ
PER_WORKER = TOTAL // NUM_WORKERS
W = 32
NBUF = 2
SKEW = NBUF - 1
NCH = PER_WORKER // W
STEADY_END = NBUF + ((NCH - NBUF) // NBUF) * NBUF


def _gather_kernel(idx_hbm, table_hbm, out_hbm, idx_v, rows, gsems, wsems):
    wid = lax.axis_index("s") * 2 + lax.axis_index("c")
    base = wid * PER_WORKER

    pltpu.sync_copy(idx_hbm.at[wid], idx_v)

    def start_gather(j, b):
        pltpu.async_copy(table_hbm.at[idx_v.at[j]], rows.at[b], gsems.at[b])

    def wait_gather(b):
        pltpu.make_async_copy(table_hbm.at[idx_v.at[0]], rows.at[b],
                              gsems.at[b]).wait()

    def start_writeback(j, b):
        pltpu.async_copy(rows.at[b], out_hbm.at[pl.ds(base + j * W, W)],
                         wsems.at[b])

    def wait_writeback(b):
        pltpu.make_async_copy(rows.at[b], out_hbm.at[pl.ds(base, W)],
                              wsems.at[b]).wait()

    def slot(j, u, first=False):
        bg = (u + SKEW) % NBUF
        if not first:
            wait_writeback(bg)
        start_gather(j + SKEW, bg)
        wait_gather(u)
        start_writeback(j, u)

    for b in range(SKEW):
        start_gather(b, b)

    for u in range(NBUF):
        slot(u, u, first=(u == 0))

    @pl.loop(NBUF, STEADY_END, step=NBUF)
    def _(i):
        for u in range(NBUF):
            slot(i + u, u)

    for j in range(STEADY_END, NCH):
        slot(j, j % NBUF)

    wait_writeback((NCH - 1) % NBUF)
    for t in range(SKEW):
        wait_gather((NCH + t) % NBUF)


def kernel(position_ids, table):
    ids = position_ids.reshape(NUM_WORKERS, NCH, W).astype(jnp.int32)
    ids = jnp.pad(ids, ((0, 0), (0, SKEW), (0, 0)))

    mesh = plsc.VectorSubcoreMesh(core_axis_name="c", subcore_axis_name="s")

    run = functools.partial(
        pl.kernel,
        mesh=mesh,
        out_type=jax.ShapeDtypeStruct((TOTAL, HIDDEN), jnp.float32),
        scratch_types=[
            pltpu.VMEM((NCH + SKEW, W), jnp.int32),
            pltpu.VMEM((NBUF, W, HIDDEN), jnp.float32),
            pltpu.SemaphoreType.DMA((NBUF,)),
            pltpu.SemaphoreType.DMA((NBUF,)),
        ],
    )(_gather_kernel)

    out = run(ids, table)
    return out.reshape(BATCH, SEQ, HIDDEN)
